# Initial kernel scaffold; baseline (speedup 1.0000x reference)
import jax
import jax.numpy as jnp
from jax.experimental import pallas as pl


def kernel(dr_state, table):
    raise NotImplementedError("write your pallas kernel here")


# speedup vs baseline: 8.1172x; 8.1172x over previous
import functools

import jax
import jax.numpy as jnp
from jax import lax
from jax.experimental import pallas as pl
from jax.experimental.pallas import tpu as pltpu, tpu_sc as plsc

_B = 16384 * 200
_D = 5
_NC, _NS = 2, 16
_NW = _NC * _NS
_BPW = _B // _NW
_C = 10240
_NCH = _BPW // _C


def _gather_body(idx_hbm, table_hbm, out_hbm, idx_v, rows_v, sem):
    wid = lax.axis_index("s") * _NC + lax.axis_index("c")

    def body(i, carry):
        base = wid * _BPW + i * _C
        pltpu.sync_copy(idx_hbm.at[pl.ds(base, _C)], idx_v)
        pltpu.async_copy(table_hbm.at[idx_v], rows_v, sem).wait()
        pltpu.sync_copy(rows_v, out_hbm.at[pl.ds(base, _C)])
        return carry

    lax.fori_loop(0, _NCH, body, 0)


@functools.lru_cache(maxsize=1)
def _build():
    mesh = plsc.VectorSubcoreMesh(core_axis_name="c", subcore_axis_name="s")
    return pl.kernel(
        _gather_body,
        out_type=jax.ShapeDtypeStruct((_B, _D), jnp.float32),
        mesh=mesh,
        scratch_types=[
            pltpu.VMEM((_C,), jnp.int32),
            pltpu.VMEM((_C, _D), jnp.float32),
            pltpu.SemaphoreType.DMA,
        ],
        compiler_params=pltpu.CompilerParams(use_tc_tiling_on_sc=False),
    )


def kernel(dr_state, table):
    flat = dr_state.reshape(-1).astype(jnp.int32)
    return _build()(flat, table)


# speedup vs baseline: 8.2630x; 1.0180x over previous
import functools

import jax
import jax.numpy as jnp
from jax import lax
from jax.experimental import pallas as pl
from jax.experimental.pallas import tpu as pltpu, tpu_sc as plsc

_B = 16384 * 200
_D = 5
_DP = 8
_NC, _NS = 2, 16
_NW = _NC * _NS
_BPW = _B // _NW
_C = 10240
_NCH = _BPW // _C


def _gather_body(idx_hbm, table_hbm, out_hbm, idx_v, rows_v, sem):
    wid = lax.axis_index("s") * _NC + lax.axis_index("c")

    def body(i, carry):
        base = wid * _BPW + i * _C
        pltpu.sync_copy(idx_hbm.at[pl.ds(base, _C)], idx_v)
        pltpu.async_copy(table_hbm.at[idx_v], rows_v, sem).wait()
        pltpu.sync_copy(rows_v, out_hbm.at[pl.ds(base, _C)])
        return carry

    lax.fori_loop(0, _NCH, body, 0)


@functools.lru_cache(maxsize=1)
def _build():
    mesh = plsc.VectorSubcoreMesh(core_axis_name="c", subcore_axis_name="s")
    return pl.kernel(
        _gather_body,
        out_type=jax.ShapeDtypeStruct((_B, _DP), jnp.float32),
        mesh=mesh,
        scratch_types=[
            pltpu.VMEM((_C,), jnp.int32),
            pltpu.VMEM((_C, _DP), jnp.float32),
            pltpu.SemaphoreType.DMA,
        ],
        compiler_params=pltpu.CompilerParams(use_tc_tiling_on_sc=False),
    )


def kernel(dr_state, table):
    flat = dr_state.reshape(-1).astype(jnp.int32)
    table_p = jnp.pad(table, ((0, 0), (0, _DP - _D)))
    return _build()(flat, table_p)[:, :_D]


# speedup vs baseline: 8.3491x; 1.0104x over previous
import functools

import jax
import jax.numpy as jnp
from jax import lax
from jax.experimental import pallas as pl
from jax.experimental.pallas import tpu as pltpu, tpu_sc as plsc

_B = 16384 * 200
_D = 5
_DP = 8
_NC, _NS = 2, 16
_NW = _NC * _NS
_BPW = _B // _NW
_C = 6400
_NCH = _BPW // _C


def _gather_body(idx_hbm, table_hbm, out_hbm,
                 idx_v0, idx_v1, rows_v0, rows_v1,
                 sem_i0, sem_i1, sem_g0, sem_g1, sem_o0, sem_o1):
    wid = lax.axis_index("s") * _NC + lax.axis_index("c")
    base0 = wid * _BPW
    idx_v = (idx_v0, idx_v1)
    rows_v = (rows_v0, rows_v1)
    sem_i = (sem_i0, sem_i1)
    sem_g = (sem_g0, sem_g1)
    sem_o = (sem_o0, sem_o1)

    def idx_dma(i):
        b = i % 2
        return pltpu.make_async_copy(
            idx_hbm.at[pl.ds(base0 + i * _C, _C)], idx_v[b], sem_i[b])

    def gather_dma(i):
        b = i % 2
        return pltpu.make_async_copy(
            table_hbm.at[idx_v[b]], rows_v[b], sem_g[b])

    def out_dma(i):
        b = i % 2
        return pltpu.make_async_copy(
            rows_v[b], out_hbm.at[pl.ds(base0 + i * _C, _C)], sem_o[b])

    idx_dma(0).start()
    idx_dma(1).start()
    idx_dma(0).wait()
    gather_dma(0).start()
    for i in range(_NCH):
        if i + 1 < _NCH:
            idx_dma(i + 1).wait()
            if i >= 1:
                out_dma(i - 1).wait()
            gather_dma(i).wait()
            gather_dma(i + 1).start()
        else:
            gather_dma(i).wait()
        out_dma(i).start()
        if i + 2 < _NCH:
            idx_dma(i + 2).start()
    out_dma(_NCH - 2).wait()
    out_dma(_NCH - 1).wait()


@functools.lru_cache(maxsize=1)
def _build():
    mesh = plsc.VectorSubcoreMesh(core_axis_name="c", subcore_axis_name="s")
    return pl.kernel(
        _gather_body,
        out_type=jax.ShapeDtypeStruct((_B, _DP), jnp.float32),
        mesh=mesh,
        scratch_types=[
            pltpu.VMEM((_C,), jnp.int32),
            pltpu.VMEM((_C,), jnp.int32),
            pltpu.VMEM((_C, _DP), jnp.float32),
            pltpu.VMEM((_C, _DP), jnp.float32),
            pltpu.SemaphoreType.DMA,
            pltpu.SemaphoreType.DMA,
            pltpu.SemaphoreType.DMA,
            pltpu.SemaphoreType.DMA,
            pltpu.SemaphoreType.DMA,
            pltpu.SemaphoreType.DMA,
        ],
        compiler_params=pltpu.CompilerParams(use_tc_tiling_on_sc=False),
    )


def kernel(dr_state, table):
    flat = dr_state.reshape(-1).astype(jnp.int32)
    table_p = jnp.pad(table, ((0, 0), (0, _DP - _D)))
    return _build()(flat, table_p)[:, :_D]
